# Initial kernel scaffold; baseline (speedup 1.0000x reference)
import jax
import jax.numpy as jnp
from jax.experimental import pallas as pl


def kernel(edge_index, coord_x, coord_y, W1, b1, W2, b2, W3, b3, W4, b4, W5, b5, Wl1, bl1, Wl2, bl2):
    raise NotImplementedError("write your pallas kernel here")


# speedup vs baseline: 5.3877x; 5.3877x over previous
import jax
import jax.numpy as jnp
from jax import lax
from jax.experimental import pallas as pl
from jax.experimental.pallas import tpu as pltpu
from jax.experimental.pallas import tpu_sc as plsc

N = 10000
E = 320000
HID = 128
HID2 = HID // 2
NC = 2
NS = 16
EB = 80
ROWS_ALL = E // EB
RPT_DEG = ROWS_ALL // NS
NP = 10240
NPT = NP // NS
RING = 5
DEG_K = 10
R_TC = 2000

NH = NP // NC
ACC_R = NH + 128
RPS = ROWS_ALL // NS
NZT = NH // NS
CH = 25
CHN = RPS // CH

f32 = jnp.float32


def _deg_body(em, ones_h, zeros_h, degs, idx_v, ones_v, acc_sh, ssem):
    cid = lax.axis_index("c")
    sid = lax.axis_index("s")
    pltpu.sync_copy(ones_h, ones_v)
    pltpu.sync_copy(zeros_h.at[pl.ds(sid * NPT, NPT)],
                    acc_sh.at[pl.ds(sid * NPT, NPT)])
    pltpu.sync_copy(em.at[cid, sid], idx_v)
    plsc.subcore_barrier()

    def outer(i, carry):
        descs = []
        for k in range(DEG_K):
            j = i * DEG_K + k
            descs.append(
                pltpu.async_copy(ones_v, acc_sh.at[idx_v.at[j]], ssem,
                                 add=True))
        for d in descs:
            d.wait()
        return carry

    lax.fori_loop(0, RPT_DEG // DEG_K, outer, 0)
    plsc.subcore_barrier()
    pltpu.sync_copy(acc_sh.at[pl.ds(sid * NPT, NPT)],
                    degs.at[pl.ds(cid * NP + sid * NPT, NPT)])


def _deg_call(em, ones1, z1):
    mesh = plsc.VectorSubcoreMesh(core_axis_name="c", subcore_axis_name="s")
    return pl.kernel(
        _deg_body,
        out_type=jax.ShapeDtypeStruct((NC * NP,), f32),
        mesh=mesh,
        scratch_types=[
            pltpu.VMEM((RPT_DEG, EB), jnp.int32),
            pltpu.VMEM((EB,), f32),
            pltpu.VMEM_SHARED((NP,), f32),
            pltpu.SemaphoreType.DMA,
        ],
    )(em, ones1, z1)


def _scat_body(em, xs, zeros_h, out, sidx_v, didx_v, rows_v, acc_sh,
               gsems, ssems):
    cid = lax.axis_index("c")
    sid = lax.axis_index("s")
    base = cid * NH
    pltpu.sync_copy(zeros_h.at[pl.ds(sid * NZT, NZT)],
                    acc_sh.at[pl.ds(sid * NZT, NZT)])
    plsc.subcore_barrier()

    def chunk_loop(c, carry0):
        pltpu.sync_copy(em.at[0, sid, c], sidx_v)
        pltpu.sync_copy(em.at[1, sid, c], didx_v)

        def remap(j, carry):
            for k in range(EB // 16):
                d = didx_v[j, pl.ds(k * 16, 16)]
                t = d - base
                inr = (d >= base) & (t < NH)
                tr = NH + (d & 127)
                didx_v[j, pl.ds(k * 16, 16)] = jnp.where(inr, t, tr)
            return carry

        lax.fori_loop(0, CH, remap, 0)

        def outer(i, carry):
            jb = i * RING
            gds = []
            for k in range(RING):
                gds.append(
                    pltpu.async_copy(xs.at[sidx_v.at[jb + k]], rows_v.at[k],
                                     gsems[k]))
            sds = []
            for k in range(RING):
                gds[k].wait()
                sds.append(
                    pltpu.async_copy(rows_v.at[k],
                                     acc_sh.at[didx_v.at[jb + k]],
                                     ssems[k], add=True))
            for d in sds:
                d.wait()
            return carry

        lax.fori_loop(0, CH // RING, outer, 0)
        return carry0

    lax.fori_loop(0, CHN, chunk_loop, 0)
    plsc.subcore_barrier()
    pltpu.sync_copy(acc_sh.at[pl.ds(sid * NZT, NZT)],
                    out.at[pl.ds(base + sid * NZT, NZT)])


def _scat_call(em, xs, zeros_h):
    mesh = plsc.VectorSubcoreMesh(core_axis_name="c", subcore_axis_name="s")

    def body(em_r, xs_r, z_r, out_r, sidx, didx, rows, acc, *sems):
        _scat_body(em_r, xs_r, z_r, out_r, sidx, didx, rows, acc,
                   sems[:RING], sems[RING:])

    return pl.kernel(
        body,
        out_type=jax.ShapeDtypeStruct((NP, HID), f32),
        mesh=mesh,
        scratch_types=[
            pltpu.VMEM((CH, EB), jnp.int32),
            pltpu.VMEM((CH, EB), jnp.int32),
            pltpu.VMEM((RING, EB, HID), f32),
            pltpu.VMEM_SHARED((ACC_R, HID), f32),
        ] + [pltpu.SemaphoreType.DMA] * (2 * RING),
    )(em, xs, zeros_h)


def _prep_body(di_ref, do_ref, cx_ref, cy_ref, xs0_ref, nd_ref, ns_ref):
    di = di_ref[...]
    nd = lax.rsqrt(jnp.maximum(di, 1.0))
    ns = lax.rsqrt(jnp.maximum(do_ref[...], 1.0))
    nd_ref[...] = nd
    ns_ref[...] = ns
    feats = jnp.concatenate(
        [di, cx_ref[...], cy_ref[...],
         jnp.zeros((R_TC, HID - 3), f32)], axis=1)
    xs0_ref[...] = feats * ns


def _prep_call(di, do, cx, cy):
    grid = (N // R_TC,)
    col = pl.BlockSpec((R_TC, 1), lambda i: (i, 0))
    return pl.pallas_call(
        _prep_body,
        grid=grid,
        in_specs=[col, col, col, col],
        out_specs=[pl.BlockSpec((R_TC, HID), lambda i: (i, 0)), col, col],
        out_shape=[
            jax.ShapeDtypeStruct((N, HID), f32),
            jax.ShapeDtypeStruct((N, 1), f32),
            jax.ShapeDtypeStruct((N, 1), f32),
        ],
    )(di, do, cx, cy)


def _mid_body(p_ref, nd_ref, ns_ref, w_ref, b_ref, xs_ref):
    agg = p_ref[...]
    t = jnp.dot(agg, w_ref[...], preferred_element_type=f32)
    h = jnp.maximum(t * nd_ref[...] + b_ref[...], 0.0)
    xs_ref[...] = h * ns_ref[...]


def _mid_call(p, nd, ns, W, b):
    grid = (N // R_TC,)
    col = pl.BlockSpec((R_TC, 1), lambda i: (i, 0))
    return pl.pallas_call(
        _mid_body,
        grid=grid,
        in_specs=[
            pl.BlockSpec((R_TC, HID), lambda i: (i, 0)),
            col, col,
            pl.BlockSpec((HID, HID), lambda i: (0, 0)),
            pl.BlockSpec((1, HID), lambda i: (0, 0)),
        ],
        out_specs=pl.BlockSpec((R_TC, HID), lambda i: (i, 0)),
        out_shape=jax.ShapeDtypeStruct((N, HID), f32),
    )(p, nd, ns, W, b)


def _final_body(p_ref, nd_ref, w_ref, b_ref, wl1_ref, bl1_ref, wl2_ref,
                bl2_ref, h_ref, emb_ref, pred_ref):
    i = pl.program_id(0)
    agg = p_ref[...]
    t = jnp.dot(agg, w_ref[...], preferred_element_type=f32)
    h = jnp.maximum(t * nd_ref[...] + b_ref[...], 0.0)
    h_ref[...] = h
    s = jnp.sum(h, axis=0, keepdims=True)

    @pl.when(i == 0)
    def _():
        emb_ref[...] = s

    @pl.when(i > 0)
    def _():
        emb_ref[...] = emb_ref[...] + s

    @pl.when(i == pl.num_programs(0) - 1)
    def _():
        emb = emb_ref[...] * (1.0 / N)
        emb_ref[...] = emb
        t1 = jnp.maximum(
            jnp.dot(emb, wl1_ref[...], preferred_element_type=f32)
            + bl1_ref[...], 0.0)
        z = jnp.dot(t1, wl2_ref[...], preferred_element_type=f32) + bl2_ref[...]
        pred_ref[...] = 1.0 / (1.0 + jnp.exp(-z))


def _final_call(p, nd, W, b, Wl1, bl1, Wl2, bl2):
    grid = (N // R_TC,)
    col = pl.BlockSpec((R_TC, 1), lambda i: (i, 0))
    full = lambda r, c: pl.BlockSpec((r, c), lambda i: (0, 0))
    return pl.pallas_call(
        _final_body,
        grid=grid,
        in_specs=[
            pl.BlockSpec((R_TC, HID), lambda i: (i, 0)),
            col,
            full(HID, HID), full(1, HID),
            full(HID, HID2), full(1, HID2),
            full(HID2, 1), full(1, 1),
        ],
        out_specs=[
            pl.BlockSpec((R_TC, HID), lambda i: (i, 0)),
            full(1, HID), full(1, 1),
        ],
        out_shape=[
            jax.ShapeDtypeStruct((N, HID), f32),
            jax.ShapeDtypeStruct((1, HID), f32),
            jax.ShapeDtypeStruct((1, 1), f32),
        ],
    )(p, nd, W, b, Wl1, bl1, Wl2, bl2)


def kernel(edge_index, coord_x, coord_y, W1, b1, W2, b2, W3, b3, W4, b4,
           W5, b5, Wl1, bl1, Wl2, bl2):
    ei = edge_index.astype(jnp.int32)
    emd = ei.reshape(2, NS, RPT_DEG, EB)
    em5 = ei.reshape(2, NS, CHN, CH, EB)
    ones1 = jnp.ones((EB,), f32)
    z1 = jnp.zeros((NP,), f32)
    z128 = jnp.zeros((NP, HID), f32)

    degs = _deg_call(emd, ones1, z1)
    deg_out = degs[:N].reshape(N, 1)
    deg_in = degs[NP:NP + N].reshape(N, 1)
    xs0, nd, ns = _prep_call(deg_in, deg_out,
                             coord_x.reshape(N, 1), coord_y.reshape(N, 1))

    W1p = jnp.zeros((HID, HID), f32).at[:3].set(W1)
    p = _scat_call(em5, xs0, z128)
    xs = _mid_call(p, nd, ns, W1p, b1.reshape(1, HID))
    for W, b in ((W2, b2), (W3, b3), (W4, b4)):
        p = _scat_call(em5, xs, z128)
        xs = _mid_call(p, nd, ns, W, b.reshape(1, HID))
    p = _scat_call(em5, xs, z128)
    h, emb, pred = _final_call(p, nd, W5, b5.reshape(1, HID),
                               Wl1, bl1.reshape(1, HID2),
                               Wl2, bl2.reshape(1, 1))
    return (pred, emb, h)


# speedup vs baseline: 8.7758x; 1.6289x over previous
import jax
import jax.numpy as jnp
from jax import lax
from jax.experimental import pallas as pl
from jax.experimental.pallas import tpu as pltpu
from jax.experimental.pallas import tpu_sc as plsc

N = 10000
E = 320000
HID = 128
HID2 = HID // 2
NC = 2
NS = 16
NW = NC * NS
EB = 80
ROWS_ALL = E // EB
RPW = ROWS_ALL // NW
RPT_DEG = ROWS_ALL // NS
NP = 10112
NPT = NP // NS
NPD = 10240
NPTD = NPD // NS
CH = 25
CHN = RPW // CH
RING = 4
DEG_K = 10
R_TC = 2000

f32 = jnp.float32


def _deg_body(em, ones_h, zeros_h, degs, idx_v, ones_v, acc_sh, ssem):
    cid = lax.axis_index("c")
    sid = lax.axis_index("s")
    pltpu.sync_copy(ones_h, ones_v)
    pltpu.sync_copy(zeros_h.at[pl.ds(sid * NPTD, NPTD)],
                    acc_sh.at[pl.ds(sid * NPTD, NPTD)])
    pltpu.sync_copy(em.at[cid, sid], idx_v)
    plsc.subcore_barrier()

    def outer(i, carry):
        descs = []
        for k in range(DEG_K):
            j = i * DEG_K + k
            descs.append(
                pltpu.async_copy(ones_v, acc_sh.at[idx_v.at[j]], ssem,
                                 add=True))
        for d in descs:
            d.wait()
        return carry

    lax.fori_loop(0, RPT_DEG // DEG_K, outer, 0)
    plsc.subcore_barrier()
    pltpu.sync_copy(acc_sh.at[pl.ds(sid * NPTD, NPTD)],
                    degs.at[pl.ds(cid * NPD + sid * NPTD, NPTD)])


def _deg_call(em, ones1, z1):
    mesh = plsc.VectorSubcoreMesh(core_axis_name="c", subcore_axis_name="s")
    return pl.kernel(
        _deg_body,
        out_type=jax.ShapeDtypeStruct((NC * NPD,), f32),
        mesh=mesh,
        scratch_types=[
            pltpu.VMEM((RPT_DEG, EB), jnp.int32),
            pltpu.VMEM((EB,), f32),
            pltpu.VMEM_SHARED((NPD,), f32),
            pltpu.SemaphoreType.DMA,
        ],
    )(em, ones1, z1)


def _scat_body(em, xs, zeros_h, out, sidx_v, didx_v, rows_v, acc_sh,
               gsems, ssems):
    cid = lax.axis_index("c")
    sid = lax.axis_index("s")
    wid = cid * NS + sid
    pltpu.sync_copy(zeros_h.at[pl.ds(sid * NPT, NPT)],
                    acc_sh.at[pl.ds(sid * NPT, NPT)])
    plsc.subcore_barrier()

    def do_batches(js):
        gds = []
        for k, j in enumerate(js):
            gds.append(
                pltpu.async_copy(xs.at[sidx_v.at[j]], rows_v.at[k],
                                 gsems[k]))
        sds = []
        for k, j in enumerate(js):
            gds[k].wait()
            sds.append(
                pltpu.async_copy(rows_v.at[k], acc_sh.at[didx_v.at[j]],
                                 ssems[k], add=True))
        for d in sds:
            d.wait()

    def chunk_loop(c, carry0):
        pltpu.sync_copy(em.at[0, wid, c], sidx_v)
        pltpu.sync_copy(em.at[1, wid, c], didx_v)

        def outer(i, carry):
            jb = i * RING
            do_batches([jb + k for k in range(RING)])
            return carry

        lax.fori_loop(0, CH // RING, outer, 0)
        do_batches([CH - 1])
        return carry0

    lax.fori_loop(0, CHN, chunk_loop, 0)
    plsc.subcore_barrier()
    pltpu.sync_copy(acc_sh.at[pl.ds(sid * NPT, NPT)],
                    out.at[cid, pl.ds(sid * NPT, NPT)])


def _scat_call(em, xs, zeros_h):
    mesh = plsc.VectorSubcoreMesh(core_axis_name="c", subcore_axis_name="s")

    def body(em_r, xs_r, z_r, out_r, sidx, didx, rows, acc, *sems):
        _scat_body(em_r, xs_r, z_r, out_r, sidx, didx, rows, acc,
                   sems[:RING], sems[RING:])

    return pl.kernel(
        body,
        out_type=jax.ShapeDtypeStruct((NC, NP, HID), f32),
        mesh=mesh,
        scratch_types=[
            pltpu.VMEM((CH, EB), jnp.int32),
            pltpu.VMEM((CH, EB), jnp.int32),
            pltpu.VMEM((RING, EB, HID), f32),
            pltpu.VMEM_SHARED((NP, HID), f32),
        ] + [pltpu.SemaphoreType.DMA] * (2 * RING),
    )(em, xs, zeros_h)


def _prep_body(di_ref, do_ref, cx_ref, cy_ref, xs0_ref, nd_ref, ns_ref):
    di = di_ref[...]
    nd = lax.rsqrt(jnp.maximum(di, 1.0))
    ns = lax.rsqrt(jnp.maximum(do_ref[...], 1.0))
    nd_ref[...] = nd
    ns_ref[...] = ns
    feats = jnp.concatenate(
        [di, cx_ref[...], cy_ref[...],
         jnp.zeros((R_TC, HID - 3), f32)], axis=1)
    xs0_ref[...] = feats * ns


def _prep_call(di, do, cx, cy):
    grid = (N // R_TC,)
    col = pl.BlockSpec((R_TC, 1), lambda i: (i, 0))
    return pl.pallas_call(
        _prep_body,
        grid=grid,
        in_specs=[col, col, col, col],
        out_specs=[pl.BlockSpec((R_TC, HID), lambda i: (i, 0)), col, col],
        out_shape=[
            jax.ShapeDtypeStruct((N, HID), f32),
            jax.ShapeDtypeStruct((N, 1), f32),
            jax.ShapeDtypeStruct((N, 1), f32),
        ],
    )(di, do, cx, cy)


def _mid_body(p_ref, nd_ref, ns_ref, w_ref, b_ref, xs_ref):
    agg = p_ref[0] + p_ref[1]
    t = jnp.dot(agg, w_ref[...], preferred_element_type=f32)
    h = jnp.maximum(t * nd_ref[...] + b_ref[...], 0.0)
    xs_ref[...] = h * ns_ref[...]


def _mid_call(p, nd, ns, W, b):
    grid = (N // R_TC,)
    col = pl.BlockSpec((R_TC, 1), lambda i: (i, 0))
    return pl.pallas_call(
        _mid_body,
        grid=grid,
        in_specs=[
            pl.BlockSpec((NC, R_TC, HID), lambda i: (0, i, 0)),
            col, col,
            pl.BlockSpec((HID, HID), lambda i: (0, 0)),
            pl.BlockSpec((1, HID), lambda i: (0, 0)),
        ],
        out_specs=pl.BlockSpec((R_TC, HID), lambda i: (i, 0)),
        out_shape=jax.ShapeDtypeStruct((N, HID), f32),
    )(p, nd, ns, W, b)


def _final_body(p_ref, nd_ref, w_ref, b_ref, wl1_ref, bl1_ref, wl2_ref,
                bl2_ref, h_ref, emb_ref, pred_ref):
    i = pl.program_id(0)
    agg = p_ref[0] + p_ref[1]
    t = jnp.dot(agg, w_ref[...], preferred_element_type=f32)
    h = jnp.maximum(t * nd_ref[...] + b_ref[...], 0.0)
    h_ref[...] = h
    s = jnp.sum(h, axis=0, keepdims=True)

    @pl.when(i == 0)
    def _():
        emb_ref[...] = s

    @pl.when(i > 0)
    def _():
        emb_ref[...] = emb_ref[...] + s

    @pl.when(i == pl.num_programs(0) - 1)
    def _():
        emb = emb_ref[...] * (1.0 / N)
        emb_ref[...] = emb
        t1 = jnp.maximum(
            jnp.dot(emb, wl1_ref[...], preferred_element_type=f32)
            + bl1_ref[...], 0.0)
        z = jnp.dot(t1, wl2_ref[...], preferred_element_type=f32) + bl2_ref[...]
        pred_ref[...] = 1.0 / (1.0 + jnp.exp(-z))


def _final_call(p, nd, W, b, Wl1, bl1, Wl2, bl2):
    grid = (N // R_TC,)
    col = pl.BlockSpec((R_TC, 1), lambda i: (i, 0))
    full = lambda r, c: pl.BlockSpec((r, c), lambda i: (0, 0))
    return pl.pallas_call(
        _final_body,
        grid=grid,
        in_specs=[
            pl.BlockSpec((NC, R_TC, HID), lambda i: (0, i, 0)),
            col,
            full(HID, HID), full(1, HID),
            full(HID, HID2), full(1, HID2),
            full(HID2, 1), full(1, 1),
        ],
        out_specs=[
            pl.BlockSpec((R_TC, HID), lambda i: (i, 0)),
            full(1, HID), full(1, 1),
        ],
        out_shape=[
            jax.ShapeDtypeStruct((N, HID), f32),
            jax.ShapeDtypeStruct((1, HID), f32),
            jax.ShapeDtypeStruct((1, 1), f32),
        ],
    )(p, nd, W, b, Wl1, bl1, Wl2, bl2)


def kernel(edge_index, coord_x, coord_y, W1, b1, W2, b2, W3, b3, W4, b4,
           W5, b5, Wl1, bl1, Wl2, bl2):
    ei = edge_index.astype(jnp.int32)
    emd = ei.reshape(2, NS, RPT_DEG, EB)
    em5 = ei.reshape(2, NW, CHN, CH, EB)
    ones1 = jnp.ones((EB,), f32)
    z1 = jnp.zeros((NPD,), f32)
    z128 = jnp.zeros((NP, HID), f32)

    degs = _deg_call(emd, ones1, z1)
    deg_out = degs[:N].reshape(N, 1)
    deg_in = degs[NPD:NPD + N].reshape(N, 1)
    xs0, nd, ns = _prep_call(deg_in, deg_out,
                             coord_x.reshape(N, 1), coord_y.reshape(N, 1))

    W1p = jnp.zeros((HID, HID), f32).at[:3].set(W1)
    p = _scat_call(em5, xs0, z128)
    xs = _mid_call(p, nd, ns, W1p, b1.reshape(1, HID))
    for W, b in ((W2, b2), (W3, b3), (W4, b4)):
        p = _scat_call(em5, xs, z128)
        xs = _mid_call(p, nd, ns, W, b.reshape(1, HID))
    p = _scat_call(em5, xs, z128)
    h, emb, pred = _final_call(p, nd, W5, b5.reshape(1, HID),
                               Wl1, bl1.reshape(1, HID2),
                               Wl2, bl2.reshape(1, 1))
    return (pred, emb, h)


# speedup vs baseline: 8.9629x; 1.0213x over previous
import jax
import jax.numpy as jnp
from jax import lax
from jax.experimental import pallas as pl
from jax.experimental.pallas import tpu as pltpu
from jax.experimental.pallas import tpu_sc as plsc

N = 10000
E = 320000
HID = 128
HID2 = HID // 2
NC = 2
NS = 16
NW = NC * NS
EB = 80
ROWS_ALL = E // EB
RPW = ROWS_ALL // NW
RPT_DEG = ROWS_ALL // NS
NP = 10112
NPT = NP // NS
NPD = 10240
NPTD = NPD // NS
CH = 25
CHN = RPW // CH
RING = 4
DEG_K = 10
R_TC = 2000

f32 = jnp.float32


def _deg_body(em, ones_h, zeros_h, degs, idx_v, ones_v, acc_sh, ssem):
    cid = lax.axis_index("c")
    sid = lax.axis_index("s")
    pltpu.sync_copy(ones_h, ones_v)
    pltpu.sync_copy(zeros_h.at[pl.ds(sid * NPTD, NPTD)],
                    acc_sh.at[pl.ds(sid * NPTD, NPTD)])
    pltpu.sync_copy(em.at[cid, sid], idx_v)
    plsc.subcore_barrier()

    def outer(i, carry):
        descs = []
        for k in range(DEG_K):
            j = i * DEG_K + k
            descs.append(
                pltpu.async_copy(ones_v, acc_sh.at[idx_v.at[j]], ssem,
                                 add=True))
        for d in descs:
            d.wait()
        return carry

    lax.fori_loop(0, RPT_DEG // DEG_K, outer, 0)
    plsc.subcore_barrier()
    pltpu.sync_copy(acc_sh.at[pl.ds(sid * NPTD, NPTD)],
                    degs.at[pl.ds(cid * NPD + sid * NPTD, NPTD)])


def _deg_call(em, ones1, z1):
    mesh = plsc.VectorSubcoreMesh(core_axis_name="c", subcore_axis_name="s")
    return pl.kernel(
        _deg_body,
        out_type=jax.ShapeDtypeStruct((NC * NPD,), f32),
        mesh=mesh,
        scratch_types=[
            pltpu.VMEM((RPT_DEG, EB), jnp.int32),
            pltpu.VMEM((EB,), f32),
            pltpu.VMEM_SHARED((NPD,), f32),
            pltpu.SemaphoreType.DMA,
        ],
    )(em, ones1, z1)


def _scat_body(em, xs, zeros_h, out, sidx_v, didx_v, rows_v, acc_sh,
               gsems, ssems):
    cid = lax.axis_index("c")
    sid = lax.axis_index("s")
    wid = cid * NS + sid
    pltpu.sync_copy(zeros_h.at[pl.ds(sid * NPT, NPT)],
                    acc_sh.at[pl.ds(sid * NPT, NPT)])
    plsc.subcore_barrier()

    def wait_scat(k):
        pltpu.make_async_copy(rows_v.at[k], acc_sh.at[didx_v.at[k]],
                              ssems[k]).wait()

    def chunk_loop(c, carry0):
        pltpu.sync_copy(em.at[0, wid, c], sidx_v)
        pltpu.sync_copy(em.at[1, wid, c], didx_v)

        def outer(i, carry):
            jb = i * RING
            @pl.when(i > 0)
            def _():
                for k in range(RING):
                    wait_scat(k)
            gds = []
            for k in range(RING):
                gds.append(
                    pltpu.async_copy(xs.at[sidx_v.at[jb + k]], rows_v.at[k],
                                     gsems[k]))
            for k in range(RING):
                gds[k].wait()
                pltpu.async_copy(rows_v.at[k], acc_sh.at[didx_v.at[jb + k]],
                                 ssems[k], add=True)
            return carry

        lax.fori_loop(0, CH // RING, outer, 0)
        wait_scat(0)
        g = pltpu.async_copy(xs.at[sidx_v.at[CH - 1]], rows_v.at[0],
                             gsems[0])
        g.wait()
        pltpu.async_copy(rows_v.at[0], acc_sh.at[didx_v.at[CH - 1]],
                         ssems[0], add=True)
        for k in range(RING):
            wait_scat(k)
        return carry0

    lax.fori_loop(0, CHN, chunk_loop, 0)
    plsc.subcore_barrier()
    pltpu.sync_copy(acc_sh.at[pl.ds(sid * NPT, NPT)],
                    out.at[cid, pl.ds(sid * NPT, NPT)])


def _scat_call(em, xs, zeros_h):
    mesh = plsc.VectorSubcoreMesh(core_axis_name="c", subcore_axis_name="s")

    def body(em_r, xs_r, z_r, out_r, sidx, didx, rows, acc, *sems):
        _scat_body(em_r, xs_r, z_r, out_r, sidx, didx, rows, acc,
                   sems[:RING], sems[RING:])

    return pl.kernel(
        body,
        out_type=jax.ShapeDtypeStruct((NC, NP, HID), f32),
        mesh=mesh,
        scratch_types=[
            pltpu.VMEM((CH, EB), jnp.int32),
            pltpu.VMEM((CH, EB), jnp.int32),
            pltpu.VMEM((RING, EB, HID), f32),
            pltpu.VMEM_SHARED((NP, HID), f32),
        ] + [pltpu.SemaphoreType.DMA] * (2 * RING),
    )(em, xs, zeros_h)


def _prep_body(di_ref, do_ref, cx_ref, cy_ref, xs0_ref, nd_ref, ns_ref):
    di = di_ref[...]
    nd = lax.rsqrt(jnp.maximum(di, 1.0))
    ns = lax.rsqrt(jnp.maximum(do_ref[...], 1.0))
    nd_ref[...] = nd
    ns_ref[...] = ns
    feats = jnp.concatenate(
        [di, cx_ref[...], cy_ref[...],
         jnp.zeros((R_TC, HID - 3), f32)], axis=1)
    xs0_ref[...] = feats * ns


def _prep_call(di, do, cx, cy):
    grid = (N // R_TC,)
    col = pl.BlockSpec((R_TC, 1), lambda i: (i, 0))
    return pl.pallas_call(
        _prep_body,
        grid=grid,
        in_specs=[col, col, col, col],
        out_specs=[pl.BlockSpec((R_TC, HID), lambda i: (i, 0)), col, col],
        out_shape=[
            jax.ShapeDtypeStruct((N, HID), f32),
            jax.ShapeDtypeStruct((N, 1), f32),
            jax.ShapeDtypeStruct((N, 1), f32),
        ],
    )(di, do, cx, cy)


def _mid_body(p_ref, nd_ref, ns_ref, w_ref, b_ref, xs_ref):
    agg = p_ref[0] + p_ref[1]
    t = jnp.dot(agg, w_ref[...], preferred_element_type=f32)
    h = jnp.maximum(t * nd_ref[...] + b_ref[...], 0.0)
    xs_ref[...] = h * ns_ref[...]


def _mid_call(p, nd, ns, W, b):
    grid = (N // R_TC,)
    col = pl.BlockSpec((R_TC, 1), lambda i: (i, 0))
    return pl.pallas_call(
        _mid_body,
        grid=grid,
        in_specs=[
            pl.BlockSpec((NC, R_TC, HID), lambda i: (0, i, 0)),
            col, col,
            pl.BlockSpec((HID, HID), lambda i: (0, 0)),
            pl.BlockSpec((1, HID), lambda i: (0, 0)),
        ],
        out_specs=pl.BlockSpec((R_TC, HID), lambda i: (i, 0)),
        out_shape=jax.ShapeDtypeStruct((N, HID), f32),
    )(p, nd, ns, W, b)


def _final_body(p_ref, nd_ref, w_ref, b_ref, wl1_ref, bl1_ref, wl2_ref,
                bl2_ref, h_ref, emb_ref, pred_ref):
    i = pl.program_id(0)
    agg = p_ref[0] + p_ref[1]
    t = jnp.dot(agg, w_ref[...], preferred_element_type=f32)
    h = jnp.maximum(t * nd_ref[...] + b_ref[...], 0.0)
    h_ref[...] = h
    s = jnp.sum(h, axis=0, keepdims=True)

    @pl.when(i == 0)
    def _():
        emb_ref[...] = s

    @pl.when(i > 0)
    def _():
        emb_ref[...] = emb_ref[...] + s

    @pl.when(i == pl.num_programs(0) - 1)
    def _():
        emb = emb_ref[...] * (1.0 / N)
        emb_ref[...] = emb
        t1 = jnp.maximum(
            jnp.dot(emb, wl1_ref[...], preferred_element_type=f32)
            + bl1_ref[...], 0.0)
        z = jnp.dot(t1, wl2_ref[...], preferred_element_type=f32) + bl2_ref[...]
        pred_ref[...] = 1.0 / (1.0 + jnp.exp(-z))


def _final_call(p, nd, W, b, Wl1, bl1, Wl2, bl2):
    grid = (N // R_TC,)
    col = pl.BlockSpec((R_TC, 1), lambda i: (i, 0))
    full = lambda r, c: pl.BlockSpec((r, c), lambda i: (0, 0))
    return pl.pallas_call(
        _final_body,
        grid=grid,
        in_specs=[
            pl.BlockSpec((NC, R_TC, HID), lambda i: (0, i, 0)),
            col,
            full(HID, HID), full(1, HID),
            full(HID, HID2), full(1, HID2),
            full(HID2, 1), full(1, 1),
        ],
        out_specs=[
            pl.BlockSpec((R_TC, HID), lambda i: (i, 0)),
            full(1, HID), full(1, 1),
        ],
        out_shape=[
            jax.ShapeDtypeStruct((N, HID), f32),
            jax.ShapeDtypeStruct((1, HID), f32),
            jax.ShapeDtypeStruct((1, 1), f32),
        ],
    )(p, nd, W, b, Wl1, bl1, Wl2, bl2)


def kernel(edge_index, coord_x, coord_y, W1, b1, W2, b2, W3, b3, W4, b4,
           W5, b5, Wl1, bl1, Wl2, bl2):
    ei = edge_index.astype(jnp.int32)
    emd = ei.reshape(2, NS, RPT_DEG, EB)
    em5 = ei.reshape(2, NW, CHN, CH, EB)
    ones1 = jnp.ones((EB,), f32)
    z1 = jnp.zeros((NPD,), f32)
    z128 = jnp.zeros((NP, HID), f32)

    degs = _deg_call(emd, ones1, z1)
    deg_out = degs[:N].reshape(N, 1)
    deg_in = degs[NPD:NPD + N].reshape(N, 1)
    xs0, nd, ns = _prep_call(deg_in, deg_out,
                             coord_x.reshape(N, 1), coord_y.reshape(N, 1))

    W1p = jnp.zeros((HID, HID), f32).at[:3].set(W1)
    p = _scat_call(em5, xs0, z128)
    xs = _mid_call(p, nd, ns, W1p, b1.reshape(1, HID))
    for W, b in ((W2, b2), (W3, b3), (W4, b4)):
        p = _scat_call(em5, xs, z128)
        xs = _mid_call(p, nd, ns, W, b.reshape(1, HID))
    p = _scat_call(em5, xs, z128)
    h, emb, pred = _final_call(p, nd, W5, b5.reshape(1, HID),
                               Wl1, bl1.reshape(1, HID2),
                               Wl2, bl2.reshape(1, 1))
    return (pred, emb, h)


# speedup vs baseline: 10.5072x; 1.1723x over previous
import jax
import jax.numpy as jnp
from jax import lax
from jax.experimental import pallas as pl
from jax.experimental.pallas import tpu as pltpu
from jax.experimental.pallas import tpu_sc as plsc

N = 10000
E = 320000
HID = 128
HID2 = HID // 2
NC = 2
NS = 16
NW = NC * NS
EB = 80
ROWS_ALL = E // EB
RPW = ROWS_ALL // NW
RPT_DEG = ROWS_ALL // NS
NP = 10112
NPT = NP // NS
NPD = 10240
NPTD = NPD // NS
CH = 25
CHN = RPW // CH
RING = 4
DEG_K = 10
R_TC = 2000

f32 = jnp.float32


def _deg_body(em, ones_h, zeros_h, degs, idx_v, ones_v, acc_sh, ssem):
    cid = lax.axis_index("c")
    sid = lax.axis_index("s")
    pltpu.sync_copy(ones_h, ones_v)
    pltpu.sync_copy(zeros_h.at[pl.ds(sid * NPTD, NPTD)],
                    acc_sh.at[pl.ds(sid * NPTD, NPTD)])
    pltpu.sync_copy(em.at[cid, sid], idx_v)
    plsc.subcore_barrier()

    def outer(i, carry):
        descs = []
        for k in range(DEG_K):
            j = i * DEG_K + k
            descs.append(
                pltpu.async_copy(ones_v, acc_sh.at[idx_v.at[j]], ssem,
                                 add=True))
        for d in descs:
            d.wait()
        return carry

    lax.fori_loop(0, RPT_DEG // DEG_K, outer, 0)
    plsc.subcore_barrier()
    pltpu.sync_copy(acc_sh.at[pl.ds(sid * NPTD, NPTD)],
                    degs.at[pl.ds(cid * NPD + sid * NPTD, NPTD)])


def _deg_call(em, ones1, z1):
    mesh = plsc.VectorSubcoreMesh(core_axis_name="c", subcore_axis_name="s")
    return pl.kernel(
        _deg_body,
        out_type=jax.ShapeDtypeStruct((NC * NPD,), f32),
        mesh=mesh,
        scratch_types=[
            pltpu.VMEM((RPT_DEG, EB), jnp.int32),
            pltpu.VMEM((EB,), f32),
            pltpu.VMEM_SHARED((NPD,), f32),
            pltpu.SemaphoreType.DMA,
        ],
    )(em, ones1, z1)


def _scat_body(em, xs, zeros_h, out, sidx_v, didx_v, rows_v, acc_sh,
               gsems, ssems):
    cid = lax.axis_index("c")
    sid = lax.axis_index("s")
    wid = cid * NS + sid
    pltpu.sync_copy(zeros_h.at[pl.ds(sid * NPT, NPT)],
                    acc_sh.at[pl.ds(sid * NPT, NPT)])
    plsc.subcore_barrier()

    def chunk_loop(c, carry0):
        pltpu.sync_copy(em.at[0, wid, c], sidx_v)
        pltpu.sync_copy(em.at[1, wid, c], didx_v)

        def step(j, carry):
            k = lax.rem(j, RING)
            kp = lax.rem(j + RING - 1, RING)

            @pl.when(j >= RING)
            def _():
                pltpu.make_async_copy(rows_v.at[k], acc_sh.at[didx_v.at[j]],
                                      ssems.at[k]).wait()

            pltpu.async_copy(xs.at[sidx_v.at[j]], rows_v.at[k], gsems.at[k])

            @pl.when(j >= 1)
            def _():
                pltpu.make_async_copy(xs.at[sidx_v.at[j - 1]], rows_v.at[kp],
                                      gsems.at[kp]).wait()
                pltpu.async_copy(rows_v.at[kp], acc_sh.at[didx_v.at[j - 1]],
                                 ssems.at[kp], add=True)

            return carry

        lax.fori_loop(0, CH, step, 0)
        kl = (CH - 1) % RING
        pltpu.make_async_copy(xs.at[sidx_v.at[CH - 1]], rows_v.at[kl],
                              gsems.at[kl]).wait()
        pltpu.async_copy(rows_v.at[kl], acc_sh.at[didx_v.at[CH - 1]],
                         ssems.at[kl], add=True)
        for k in range(RING):
            pltpu.make_async_copy(rows_v.at[k], acc_sh.at[didx_v.at[k]],
                                  ssems.at[k]).wait()
        return carry0

    lax.fori_loop(0, CHN, chunk_loop, 0)
    plsc.subcore_barrier()
    pltpu.sync_copy(acc_sh.at[pl.ds(sid * NPT, NPT)],
                    out.at[cid, pl.ds(sid * NPT, NPT)])


def _scat_call(em, xs, zeros_h):
    mesh = plsc.VectorSubcoreMesh(core_axis_name="c", subcore_axis_name="s")

    return pl.kernel(
        _scat_body,
        out_type=jax.ShapeDtypeStruct((NC, NP, HID), f32),
        mesh=mesh,
        scratch_types=[
            pltpu.VMEM((CH, EB), jnp.int32),
            pltpu.VMEM((CH, EB), jnp.int32),
            pltpu.VMEM((RING, EB, HID), f32),
            pltpu.VMEM_SHARED((NP, HID), f32),
            pltpu.SemaphoreType.DMA((RING,)),
            pltpu.SemaphoreType.DMA((RING,)),
        ],
    )(em, xs, zeros_h)


def _prep_body(di_ref, do_ref, cx_ref, cy_ref, xs0_ref, nd_ref, ns_ref):
    di = di_ref[...]
    nd = lax.rsqrt(jnp.maximum(di, 1.0))
    ns = lax.rsqrt(jnp.maximum(do_ref[...], 1.0))
    nd_ref[...] = nd
    ns_ref[...] = ns
    feats = jnp.concatenate(
        [di, cx_ref[...], cy_ref[...],
         jnp.zeros((R_TC, HID - 3), f32)], axis=1)
    xs0_ref[...] = feats * ns


def _prep_call(di, do, cx, cy):
    grid = (N // R_TC,)
    col = pl.BlockSpec((R_TC, 1), lambda i: (i, 0))
    return pl.pallas_call(
        _prep_body,
        grid=grid,
        in_specs=[col, col, col, col],
        out_specs=[pl.BlockSpec((R_TC, HID), lambda i: (i, 0)), col, col],
        out_shape=[
            jax.ShapeDtypeStruct((N, HID), f32),
            jax.ShapeDtypeStruct((N, 1), f32),
            jax.ShapeDtypeStruct((N, 1), f32),
        ],
    )(di, do, cx, cy)


def _mid_body(p_ref, nd_ref, ns_ref, w_ref, b_ref, xs_ref):
    agg = p_ref[0] + p_ref[1]
    t = jnp.dot(agg, w_ref[...], preferred_element_type=f32)
    h = jnp.maximum(t * nd_ref[...] + b_ref[...], 0.0)
    xs_ref[...] = h * ns_ref[...]


def _mid_call(p, nd, ns, W, b):
    grid = (N // R_TC,)
    col = pl.BlockSpec((R_TC, 1), lambda i: (i, 0))
    return pl.pallas_call(
        _mid_body,
        grid=grid,
        in_specs=[
            pl.BlockSpec((NC, R_TC, HID), lambda i: (0, i, 0)),
            col, col,
            pl.BlockSpec((HID, HID), lambda i: (0, 0)),
            pl.BlockSpec((1, HID), lambda i: (0, 0)),
        ],
        out_specs=pl.BlockSpec((R_TC, HID), lambda i: (i, 0)),
        out_shape=jax.ShapeDtypeStruct((N, HID), f32),
    )(p, nd, ns, W, b)


def _final_body(p_ref, nd_ref, w_ref, b_ref, wl1_ref, bl1_ref, wl2_ref,
                bl2_ref, h_ref, emb_ref, pred_ref):
    i = pl.program_id(0)
    agg = p_ref[0] + p_ref[1]
    t = jnp.dot(agg, w_ref[...], preferred_element_type=f32)
    h = jnp.maximum(t * nd_ref[...] + b_ref[...], 0.0)
    h_ref[...] = h
    s = jnp.sum(h, axis=0, keepdims=True)

    @pl.when(i == 0)
    def _():
        emb_ref[...] = s

    @pl.when(i > 0)
    def _():
        emb_ref[...] = emb_ref[...] + s

    @pl.when(i == pl.num_programs(0) - 1)
    def _():
        emb = emb_ref[...] * (1.0 / N)
        emb_ref[...] = emb
        t1 = jnp.maximum(
            jnp.dot(emb, wl1_ref[...], preferred_element_type=f32)
            + bl1_ref[...], 0.0)
        z = jnp.dot(t1, wl2_ref[...], preferred_element_type=f32) + bl2_ref[...]
        pred_ref[...] = 1.0 / (1.0 + jnp.exp(-z))


def _final_call(p, nd, W, b, Wl1, bl1, Wl2, bl2):
    grid = (N // R_TC,)
    col = pl.BlockSpec((R_TC, 1), lambda i: (i, 0))
    full = lambda r, c: pl.BlockSpec((r, c), lambda i: (0, 0))
    return pl.pallas_call(
        _final_body,
        grid=grid,
        in_specs=[
            pl.BlockSpec((NC, R_TC, HID), lambda i: (0, i, 0)),
            col,
            full(HID, HID), full(1, HID),
            full(HID, HID2), full(1, HID2),
            full(HID2, 1), full(1, 1),
        ],
        out_specs=[
            pl.BlockSpec((R_TC, HID), lambda i: (i, 0)),
            full(1, HID), full(1, 1),
        ],
        out_shape=[
            jax.ShapeDtypeStruct((N, HID), f32),
            jax.ShapeDtypeStruct((1, HID), f32),
            jax.ShapeDtypeStruct((1, 1), f32),
        ],
    )(p, nd, W, b, Wl1, bl1, Wl2, bl2)


def kernel(edge_index, coord_x, coord_y, W1, b1, W2, b2, W3, b3, W4, b4,
           W5, b5, Wl1, bl1, Wl2, bl2):
    ei = edge_index.astype(jnp.int32)
    emd = ei.reshape(2, NS, RPT_DEG, EB)
    em5 = ei.reshape(2, NW, CHN, CH, EB)
    ones1 = jnp.ones((EB,), f32)
    z1 = jnp.zeros((NPD,), f32)
    z128 = jnp.zeros((NP, HID), f32)

    degs = _deg_call(emd, ones1, z1)
    deg_out = degs[:N].reshape(N, 1)
    deg_in = degs[NPD:NPD + N].reshape(N, 1)
    xs0, nd, ns = _prep_call(deg_in, deg_out,
                             coord_x.reshape(N, 1), coord_y.reshape(N, 1))

    W1p = jnp.zeros((HID, HID), f32).at[:3].set(W1)
    p = _scat_call(em5, xs0, z128)
    xs = _mid_call(p, nd, ns, W1p, b1.reshape(1, HID))
    for W, b in ((W2, b2), (W3, b3), (W4, b4)):
        p = _scat_call(em5, xs, z128)
        xs = _mid_call(p, nd, ns, W, b.reshape(1, HID))
    p = _scat_call(em5, xs, z128)
    h, emb, pred = _final_call(p, nd, W5, b5.reshape(1, HID),
                               Wl1, bl1.reshape(1, HID2),
                               Wl2, bl2.reshape(1, 1))
    return (pred, emb, h)


# speedup vs baseline: 11.8354x; 1.1264x over previous
import jax
import jax.numpy as jnp
from jax import lax
from jax.experimental import pallas as pl
from jax.experimental.pallas import tpu as pltpu
from jax.experimental.pallas import tpu_sc as plsc

N = 10000
E = 320000
HID = 128
HID2 = HID // 2
NC = 2
NS = 16
NW = NC * NS
EB = 80
EBF = 100
ROWS_ALL = E // EB
ROWS_F = E // EBF
RPW = ROWS_F // NW
RPT_DEG = ROWS_ALL // NS
NP = 10112
NPT = NP // NS
NPD = 10240
NPTD = NPD // NS
CH = 25
CHN = RPW // CH
RING = 3
DEG_K = 10
R_TC = 2000

f32 = jnp.float32


def _deg_body(em, ones_h, zeros_h, degs, idx_v, ones_v, acc_sh, ssem):
    cid = lax.axis_index("c")
    sid = lax.axis_index("s")
    pltpu.sync_copy(ones_h, ones_v)
    pltpu.sync_copy(zeros_h.at[pl.ds(sid * NPTD, NPTD)],
                    acc_sh.at[pl.ds(sid * NPTD, NPTD)])
    pltpu.sync_copy(em.at[cid, sid], idx_v)
    plsc.subcore_barrier()

    def outer(i, carry):
        descs = []
        for k in range(DEG_K):
            j = i * DEG_K + k
            descs.append(
                pltpu.async_copy(ones_v, acc_sh.at[idx_v.at[j]], ssem,
                                 add=True))
        for d in descs:
            d.wait()
        return carry

    lax.fori_loop(0, RPT_DEG // DEG_K, outer, 0)
    plsc.subcore_barrier()
    pltpu.sync_copy(acc_sh.at[pl.ds(sid * NPTD, NPTD)],
                    degs.at[pl.ds(cid * NPD + sid * NPTD, NPTD)])


def _deg_call(em, ones1, z1):
    mesh = plsc.VectorSubcoreMesh(core_axis_name="c", subcore_axis_name="s")
    return pl.kernel(
        _deg_body,
        out_type=jax.ShapeDtypeStruct((NC * NPD,), f32),
        mesh=mesh,
        scratch_types=[
            pltpu.VMEM((RPT_DEG, EB), jnp.int32),
            pltpu.VMEM((EB,), f32),
            pltpu.VMEM_SHARED((NPD,), f32),
            pltpu.SemaphoreType.DMA,
        ],
    )(em, ones1, z1)


def _scat_body(em, xs, zeros_h, out, sidx_v, didx_v, rows_v, acc_sh,
               gsems, ssems):
    cid = lax.axis_index("c")
    sid = lax.axis_index("s")
    wid = cid * NS + sid
    pltpu.sync_copy(zeros_h.at[pl.ds(sid * NPT, NPT)],
                    acc_sh.at[pl.ds(sid * NPT, NPT)])
    plsc.subcore_barrier()

    def chunk_loop(c, carry0):
        pltpu.sync_copy(em.at[0, wid, c], sidx_v)
        pltpu.sync_copy(em.at[1, wid, c], didx_v)

        def step(j, carry):
            k = lax.rem(j, RING)
            kp = lax.rem(j + RING - 1, RING)

            @pl.when(j >= RING)
            def _():
                pltpu.make_async_copy(rows_v.at[k], acc_sh.at[didx_v.at[j]],
                                      ssems.at[k]).wait()

            pltpu.async_copy(xs.at[sidx_v.at[j]], rows_v.at[k], gsems.at[k])

            @pl.when(j >= 1)
            def _():
                pltpu.make_async_copy(xs.at[sidx_v.at[j - 1]], rows_v.at[kp],
                                      gsems.at[kp]).wait()
                pltpu.async_copy(rows_v.at[kp], acc_sh.at[didx_v.at[j - 1]],
                                 ssems.at[kp], add=True)

            return carry

        lax.fori_loop(0, CH, step, 0)
        kl = (CH - 1) % RING
        pltpu.make_async_copy(xs.at[sidx_v.at[CH - 1]], rows_v.at[kl],
                              gsems.at[kl]).wait()
        pltpu.async_copy(rows_v.at[kl], acc_sh.at[didx_v.at[CH - 1]],
                         ssems.at[kl], add=True)
        for k in range(RING):
            pltpu.make_async_copy(rows_v.at[k], acc_sh.at[didx_v.at[k]],
                                  ssems.at[k]).wait()
        return carry0

    lax.fori_loop(0, CHN, chunk_loop, 0)
    plsc.subcore_barrier()
    pltpu.sync_copy(acc_sh.at[pl.ds(sid * NPT, NPT)],
                    out.at[cid, pl.ds(sid * NPT, NPT)])


def _scat_call(em, xs, zeros_h):
    mesh = plsc.VectorSubcoreMesh(core_axis_name="c", subcore_axis_name="s")

    return pl.kernel(
        _scat_body,
        out_type=jax.ShapeDtypeStruct((NC, NP, HID), f32),
        mesh=mesh,
        scratch_types=[
            pltpu.VMEM((CH, EBF), jnp.int32),
            pltpu.VMEM((CH, EBF), jnp.int32),
            pltpu.VMEM((RING, EBF, HID), f32),
            pltpu.VMEM_SHARED((NP, HID), f32),
            pltpu.SemaphoreType.DMA((RING,)),
            pltpu.SemaphoreType.DMA((RING,)),
        ],
    )(em, xs, zeros_h)


def _prep_body(di_ref, do_ref, cx_ref, cy_ref, xs0_ref, nd_ref, ns_ref):
    di = di_ref[...]
    nd = lax.rsqrt(jnp.maximum(di, 1.0))
    ns = lax.rsqrt(jnp.maximum(do_ref[...], 1.0))
    nd_ref[...] = nd
    ns_ref[...] = ns
    feats = jnp.concatenate(
        [di, cx_ref[...], cy_ref[...],
         jnp.zeros((R_TC, HID - 3), f32)], axis=1)
    xs0_ref[...] = feats * ns


def _prep_call(di, do, cx, cy):
    grid = (N // R_TC,)
    col = pl.BlockSpec((R_TC, 1), lambda i: (i, 0))
    return pl.pallas_call(
        _prep_body,
        grid=grid,
        in_specs=[col, col, col, col],
        out_specs=[pl.BlockSpec((R_TC, HID), lambda i: (i, 0)), col, col],
        out_shape=[
            jax.ShapeDtypeStruct((N, HID), f32),
            jax.ShapeDtypeStruct((N, 1), f32),
            jax.ShapeDtypeStruct((N, 1), f32),
        ],
    )(di, do, cx, cy)


def _mid_body(p_ref, nd_ref, ns_ref, w_ref, b_ref, xs_ref):
    agg = p_ref[0] + p_ref[1]
    t = jnp.dot(agg, w_ref[...], preferred_element_type=f32)
    h = jnp.maximum(t * nd_ref[...] + b_ref[...], 0.0)
    xs_ref[...] = h * ns_ref[...]


def _mid_call(p, nd, ns, W, b):
    grid = (N // R_TC,)
    col = pl.BlockSpec((R_TC, 1), lambda i: (i, 0))
    return pl.pallas_call(
        _mid_body,
        grid=grid,
        in_specs=[
            pl.BlockSpec((NC, R_TC, HID), lambda i: (0, i, 0)),
            col, col,
            pl.BlockSpec((HID, HID), lambda i: (0, 0)),
            pl.BlockSpec((1, HID), lambda i: (0, 0)),
        ],
        out_specs=pl.BlockSpec((R_TC, HID), lambda i: (i, 0)),
        out_shape=jax.ShapeDtypeStruct((N, HID), f32),
    )(p, nd, ns, W, b)


def _final_body(p_ref, nd_ref, w_ref, b_ref, wl1_ref, bl1_ref, wl2_ref,
                bl2_ref, h_ref, emb_ref, pred_ref):
    i = pl.program_id(0)
    agg = p_ref[0] + p_ref[1]
    t = jnp.dot(agg, w_ref[...], preferred_element_type=f32)
    h = jnp.maximum(t * nd_ref[...] + b_ref[...], 0.0)
    h_ref[...] = h
    s = jnp.sum(h, axis=0, keepdims=True)

    @pl.when(i == 0)
    def _():
        emb_ref[...] = s

    @pl.when(i > 0)
    def _():
        emb_ref[...] = emb_ref[...] + s

    @pl.when(i == pl.num_programs(0) - 1)
    def _():
        emb = emb_ref[...] * (1.0 / N)
        emb_ref[...] = emb
        t1 = jnp.maximum(
            jnp.dot(emb, wl1_ref[...], preferred_element_type=f32)
            + bl1_ref[...], 0.0)
        z = jnp.dot(t1, wl2_ref[...], preferred_element_type=f32) + bl2_ref[...]
        pred_ref[...] = 1.0 / (1.0 + jnp.exp(-z))


def _final_call(p, nd, W, b, Wl1, bl1, Wl2, bl2):
    grid = (N // R_TC,)
    col = pl.BlockSpec((R_TC, 1), lambda i: (i, 0))
    full = lambda r, c: pl.BlockSpec((r, c), lambda i: (0, 0))
    return pl.pallas_call(
        _final_body,
        grid=grid,
        in_specs=[
            pl.BlockSpec((NC, R_TC, HID), lambda i: (0, i, 0)),
            col,
            full(HID, HID), full(1, HID),
            full(HID, HID2), full(1, HID2),
            full(HID2, 1), full(1, 1),
        ],
        out_specs=[
            pl.BlockSpec((R_TC, HID), lambda i: (i, 0)),
            full(1, HID), full(1, 1),
        ],
        out_shape=[
            jax.ShapeDtypeStruct((N, HID), f32),
            jax.ShapeDtypeStruct((1, HID), f32),
            jax.ShapeDtypeStruct((1, 1), f32),
        ],
    )(p, nd, W, b, Wl1, bl1, Wl2, bl2)


def kernel(edge_index, coord_x, coord_y, W1, b1, W2, b2, W3, b3, W4, b4,
           W5, b5, Wl1, bl1, Wl2, bl2):
    ei = edge_index.astype(jnp.int32)
    emd = ei.reshape(2, NS, RPT_DEG, EB)
    em5 = ei.reshape(2, NW, CHN, CH, EBF)
    ones1 = jnp.ones((EB,), f32)
    z1 = jnp.zeros((NPD,), f32)
    z128 = jnp.zeros((NP, HID), f32)

    degs = _deg_call(emd, ones1, z1)
    deg_out = degs[:N].reshape(N, 1)
    deg_in = degs[NPD:NPD + N].reshape(N, 1)
    xs0, nd, ns = _prep_call(deg_in, deg_out,
                             coord_x.reshape(N, 1), coord_y.reshape(N, 1))

    W1p = jnp.zeros((HID, HID), f32).at[:3].set(W1)
    p = _scat_call(em5, xs0, z128)
    xs = _mid_call(p, nd, ns, W1p, b1.reshape(1, HID))
    for W, b in ((W2, b2), (W3, b3), (W4, b4)):
        p = _scat_call(em5, xs, z128)
        xs = _mid_call(p, nd, ns, W, b.reshape(1, HID))
    p = _scat_call(em5, xs, z128)
    h, emb, pred = _final_call(p, nd, W5, b5.reshape(1, HID),
                               Wl1, bl1.reshape(1, HID2),
                               Wl2, bl2.reshape(1, 1))
    return (pred, emb, h)
